# baseline (speedup 1.0000x reference)
import jax
import jax.numpy as jnp
from jax import lax
from jax.experimental import pallas as pl
from jax.experimental.pallas import tpu as pltpu
from jax.experimental.pallas import tpu_sc as plsc

N = 10000
FIN = 128
HEADS = 8
OC = 16
HID = HEADS * OC
NCLS = 40

NA = 10112
RB = 128
NW = 32
NSUB = 16
ROWS_PT = NA // NSUB

C1 = 80
K1 = 132
C2 = 80
K2 = 132
T = C1 * K1
ET_PAD = NW * T

W1ACC = HID + 16
TW1 = 160
GW = 48
W2ACC = 64
TW2 = 96

_f32 = jnp.float32
_i32 = jnp.int32
_i16 = jnp.int16
_bf16 = jnp.bfloat16


def _tc1_body(x_ref, w_ref, es_ref, ed_ref, h_ref, s_ref, d_ref):
    h = jnp.dot(x_ref[...], w_ref[...], preferred_element_type=_f32)
    h_ref[...] = h.astype(_bf16)
    s_ref[...] = jnp.dot(h, es_ref[...], preferred_element_type=_f32)
    d_ref[...] = jnp.dot(h, ed_ref[...], preferred_element_type=_f32)


def _tc2_body(p0_ref, p1_ref, b1_ref, w2_ref, es_ref, ed_ref, r_ref,
              g_ref, s_ref, d_ref):
    acc = p0_ref[...] + p1_ref[...]
    num = acc[:, :HID]
    den = acc[:, HID:HID + HEADS]
    dexp = jnp.dot(den, r_ref[...], preferred_element_type=_f32,
                   precision=lax.Precision.HIGHEST)
    o1 = num / (dexp + 1e-16) + b1_ref[...]
    h2 = jnp.where(o1 > 0.0, o1, jnp.exp(o1) - 1.0)
    g = jnp.dot(h2, w2_ref[...], preferred_element_type=_f32)
    g_ref[...] = g
    s_ref[...] = jnp.dot(g, es_ref[...], preferred_element_type=_f32)
    d_ref[...] = jnp.dot(g, ed_ref[...], preferred_element_type=_f32)


def _tc3_body(q0_ref, q1_ref, b2_ref, o_ref):
    acc = q0_ref[...] + q1_ref[...]
    den = acc[:, GW:GW + 1]
    o = acc / (den + 1e-16) + b2_ref[...]
    col = lax.broadcasted_iota(_i32, (RB, W2ACC), 1)
    valid = col < NCLS
    ov = jnp.where(valid, o, -1e30)
    m = jnp.max(ov, axis=1, keepdims=True)
    ex = jnp.where(valid, jnp.exp(ov - m), 0.0)
    ssum = jnp.sum(ex, axis=1, keepdims=True)
    o_ref[...] = (o - m) - jnp.log(ssum)


def _edge_weight(a):
    a = jnp.where(a >= 0.0, a, 0.2 * a)
    return jnp.exp(a)


def _edge1(tb, db, msgb, b, e):
    sv = plsc.bitcast(tb[b, e, pl.ds(HID, 32)], _f32)
    a = sv + db[b, e, :]
    w = _edge_weight(a)
    pairs = [
        plsc.unpack(plsc.bitcast(tb[b, e, pl.ds(32 * p, 32)], _bf16),
                    format=plsc.PackFormat.INTERLEAVED)
        for p in range(4)
    ]
    msgb[b, e, pl.ds(HID, 16)] = w
    prods = []
    for p in range(4):
        fa, fb = pairs[p]
        prods.append(fa * w[2 * p])
        prods.append(fb * w[2 * p + 1])
    for f in range(8):
        msgb[b, e, pl.ds(f * 16, 16)] = prods[f]


def _edge2(tb, db, msgb, b, e):
    sv = plsc.bitcast(tb[b, e, pl.ds(64, 32)], _f32)
    a = sv + db[b, e, :]
    w = _edge_weight(a)
    fa, fb = plsc.unpack(plsc.bitcast(tb[b, e, pl.ds(0, 32)], _bf16),
                         format=plsc.PackFormat.INTERLEAVED)
    f2 = plsc.bitcast(tb[b, e, pl.ds(32, 32)], _f32)
    msgb[b, e, pl.ds(GW, 16)] = w
    pa = fa * w[0]
    pb = fb * w[0]
    pc = f2 * w[0]
    msgb[b, e, pl.ds(0, 16)] = pa
    msgb[b, e, pl.ds(16, 16)] = pb
    msgb[b, e, pl.ds(32, 16)] = pc


def _make_sc_body(CC, KK, W, edge_fn):
    ZR = ROWS_PT // 8

    def body(sdidx_h, tab, dtab, out,
             idxb, tb, db, msgb, accum, isem, gs0, gs1, ss0, ss1):
        cid = lax.axis_index("c")
        sid = lax.axis_index("s")
        t = cid * NSUB + sid
        gsem = (gs0, gs1)
        ssem = (ss0, ss1)

        def zrow(r, c2):
            for ccol in range(W // 16):
                msgb[0, r, pl.ds(ccol * 16, 16)] = jnp.zeros((16,), _f32)
            return c2

        lax.fori_loop(0, ZR, zrow, 0)
        for i in range(8):
            pltpu.sync_copy(msgb.at[0, pl.ds(0, ZR)],
                            accum.at[pl.ds(sid * ROWS_PT + i * ZR, ZR)])
        pltpu.sync_copy(sdidx_h.at[t, 0], idxb.at[0])
        pltpu.sync_copy(sdidx_h.at[t, 1], idxb.at[1])
        plsc.subcore_barrier()

        def issue_gathers(islot, dslot):
            pltpu.async_copy(tab.at[idxb.at[islot, 0]], tb.at[dslot],
                             gsem[dslot])
            pltpu.async_copy(dtab.at[idxb.at[islot, 1]], db.at[dslot],
                             gsem[dslot])

        issue_gathers(0, 0)

        def outer(qo, carry):
            for q in range(4):
                j = qo * 4 + q
                b = q & 1

                @pl.when((j >= 1) & (j <= KK - 2))
                def _():
                    pltpu.make_async_copy(sdidx_h.at[t, 0],
                                          idxb.at[(q + 1) % 4], isem).wait()

                @pl.when(j <= KK - 2)
                def _():
                    issue_gathers((q + 1) % 4, (q + 1) & 1)

                pltpu.make_async_copy(tab.at[idxb.at[q, 0]], tb.at[b],
                                      gsem[b]).wait()
                pltpu.make_async_copy(dtab.at[idxb.at[q, 1]], db.at[b],
                                      gsem[b]).wait()

                @pl.when(j >= 2)
                def _():
                    pltpu.make_async_copy(
                        msgb.at[b], accum.at[idxb.at[(q + 2) % 4, 1]],
                        ssem[b]).wait()

                @pl.when(j <= KK - 3)
                def _():
                    pltpu.async_copy(sdidx_h.at[t, j + 2],
                                     idxb.at[(q + 2) % 4], isem)

                def edge(e, c2):
                    edge_fn(tb, db, msgb, b, e)
                    return c2

                lax.fori_loop(0, CC, edge, 0, unroll=4)
                pltpu.async_copy(msgb.at[b], accum.at[idxb.at[q, 1]],
                                 ssem[b], add=True)
            return carry

        lax.fori_loop(0, KK // 4, outer, 0)
        pltpu.make_async_copy(msgb.at[0], accum.at[idxb.at[(KK - 2) % 4, 1]],
                              ss0).wait()
        pltpu.make_async_copy(msgb.at[1], accum.at[idxb.at[(KK - 1) % 4, 1]],
                              ss1).wait()
        plsc.subcore_barrier()
        pltpu.sync_copy(accum.at[pl.ds(sid * ROWS_PT, ROWS_PT)],
                        out.at[cid, pl.ds(sid * ROWS_PT, ROWS_PT)])

    return body


_sc1_body = _make_sc_body(C1, K1, W1ACC, _edge1)
_sc2_body = _make_sc_body(C2, K2, W2ACC, _edge2)


def _sc_mesh():
    return plsc.VectorSubcoreMesh(core_axis_name="c", subcore_axis_name="s",
                                  num_cores=2, num_subcores=NSUB)


def _sc_params():
    return pltpu.CompilerParams(needs_layout_passes=False,
                                use_tc_tiling_on_sc=False)


def _sc_scratch(CC, TW, W):
    return [
        pltpu.VMEM((4, 2, CC), _i32),
        pltpu.VMEM((2, CC, TW), _i16),
        pltpu.VMEM((2, CC, 16), _f32),
        pltpu.VMEM((2, CC, W), _f32),
        pltpu.VMEM_SHARED((NA, W), _f32),
        pltpu.SemaphoreType.DMA,
        pltpu.SemaphoreType.DMA,
        pltpu.SemaphoreType.DMA,
        pltpu.SemaphoreType.DMA,
        pltpu.SemaphoreType.DMA,
    ]


def kernel(x, edge_index, W1, a_src1, a_dst1, b1, W2, a_src2, a_dst2, b2):
    loops = jnp.arange(N, dtype=_i32)
    src = jnp.concatenate([edge_index[0].astype(_i32), loops])
    dst = jnp.concatenate([edge_index[1].astype(_i32), loops])
    padlen = ET_PAD - src.shape[0]
    pad = jnp.full((padlen,), N, _i32)
    src_f = jnp.concatenate([src, pad])
    dst_f = jnp.concatenate([dst, pad])
    sd1 = jnp.stack([src_f.reshape(NW, K1, C1),
                     dst_f.reshape(NW, K1, C1)], axis=2)
    sd2 = sd1

    x_pad = jnp.zeros((NA, FIN), _f32).at[:N].set(x)

    hh = jnp.arange(HID, dtype=_i32) // OC
    kk = jnp.arange(16, dtype=_i32) % HEADS
    k8 = jnp.arange(HEADS, dtype=_i32)
    es1 = jnp.where(hh[:, None] == kk[None, :], a_src1.reshape(-1)[:, None], 0.0)
    ed1 = jnp.where(hh[:, None] == kk[None, :], a_dst1.reshape(-1)[:, None], 0.0)
    rmat = (hh[None, :] == k8[:, None]).astype(_f32)
    w2p = jnp.zeros((HID, GW), _f32).at[:, :NCLS].set(W2)
    a2s = jnp.zeros((GW,), _f32).at[:NCLS].set(a_src2[0])
    a2d = jnp.zeros((GW,), _f32).at[:NCLS].set(a_dst2[0])
    es2 = jnp.broadcast_to(a2s[:, None], (GW, 16))
    ed2 = jnp.broadcast_to(a2d[:, None], (GW, 16))
    b1r = b1.reshape(1, HID)
    b2r = jnp.zeros((1, W2ACC), _f32).at[0, :NCLS].set(b2)

    grid = (NA // RB,)
    rep = lambda i: (0, 0)
    row = lambda i: (i, 0)

    hb1, s1, dd1 = pl.pallas_call(
        _tc1_body,
        grid=grid,
        in_specs=[pl.BlockSpec((RB, FIN), row),
                  pl.BlockSpec((FIN, HID), rep),
                  pl.BlockSpec((HID, 16), rep),
                  pl.BlockSpec((HID, 16), rep)],
        out_specs=[pl.BlockSpec((RB, HID), row),
                   pl.BlockSpec((RB, 16), row),
                   pl.BlockSpec((RB, 16), row)],
        out_shape=[jax.ShapeDtypeStruct((NA, HID), _bf16),
                   jax.ShapeDtypeStruct((NA, 16), _f32),
                   jax.ShapeDtypeStruct((NA, 16), _f32)],
    )(x_pad, W1, es1, ed1)

    hi1 = hb1.reshape(NA, 4, 2, 16).transpose(0, 1, 3, 2).reshape(NA, HID)
    tab1 = jnp.concatenate(
        [lax.bitcast_convert_type(hi1, _i16),
         lax.bitcast_convert_type(s1, _i16).reshape(NA, 32)], axis=1)

    parts1 = pl.kernel(
        _sc1_body,
        out_type=jax.ShapeDtypeStruct((2, NA, W1ACC), _f32),
        mesh=_sc_mesh(),
        scratch_types=_sc_scratch(C1, TW1, W1ACC),
        compiler_params=_sc_params(),
    )(sd1, tab1, dd1)

    g2, s2, dd2 = pl.pallas_call(
        _tc2_body,
        grid=grid,
        in_specs=[pl.BlockSpec((RB, W1ACC), row),
                  pl.BlockSpec((RB, W1ACC), row),
                  pl.BlockSpec((1, HID), rep),
                  pl.BlockSpec((FIN, GW), rep),
                  pl.BlockSpec((GW, 16), rep),
                  pl.BlockSpec((GW, 16), rep),
                  pl.BlockSpec((HEADS, HID), rep)],
        out_specs=[pl.BlockSpec((RB, GW), row),
                   pl.BlockSpec((RB, 16), row),
                   pl.BlockSpec((RB, 16), row)],
        out_shape=[jax.ShapeDtypeStruct((NA, GW), _f32),
                   jax.ShapeDtypeStruct((NA, 16), _f32),
                   jax.ShapeDtypeStruct((NA, 16), _f32)],
    )(parts1[0], parts1[1], b1r, w2p, es2, ed2, rmat)

    gb = g2[:, :32].astype(_bf16)
    gi = gb.reshape(NA, 1, 2, 16).transpose(0, 1, 3, 2).reshape(NA, 32)
    tab2 = jnp.concatenate(
        [lax.bitcast_convert_type(gi, _i16),
         lax.bitcast_convert_type(g2[:, 32:GW], _i16).reshape(NA, 32),
         lax.bitcast_convert_type(s2, _i16).reshape(NA, 32)], axis=1)

    parts2 = pl.kernel(
        _sc2_body,
        out_type=jax.ShapeDtypeStruct((2, NA, W2ACC), _f32),
        mesh=_sc_mesh(),
        scratch_types=_sc_scratch(C2, TW2, W2ACC),
        compiler_params=_sc_params(),
    )(sd2, tab2, dd2)

    res = pl.pallas_call(
        _tc3_body,
        grid=grid,
        in_specs=[pl.BlockSpec((RB, W2ACC), row),
                  pl.BlockSpec((RB, W2ACC), row),
                  pl.BlockSpec((1, W2ACC), rep)],
        out_specs=pl.BlockSpec((RB, W2ACC), row),
        out_shape=jax.ShapeDtypeStruct((NA, W2ACC), _f32),
    )(parts2[0], parts2[1], b2r)

    return res[:N, :NCLS]


# speedup vs baseline: 1.0609x; 1.0609x over previous
import jax
import jax.numpy as jnp
from jax import lax
from jax.experimental import pallas as pl
from jax.experimental.pallas import tpu as pltpu
from jax.experimental.pallas import tpu_sc as plsc

N = 10000
FIN = 128
HEADS = 8
OC = 16
HID = HEADS * OC
NCLS = 40

NA = 10112
RB = 128
NW = 32
NSUB = 16
ROWS_PT = NA // NSUB

C1 = 80
K1 = 132
C2 = 120
K2 = 88
T = C1 * K1
ET_PAD = NW * T

W1ACC = HID + 16
TW1 = 160
GW = 48
W2ACC = 64
TW2 = 96

_f32 = jnp.float32
_i32 = jnp.int32
_i16 = jnp.int16
_bf16 = jnp.bfloat16


def _tc1_body(x_ref, w_ref, es_ref, ed_ref, h_ref, s_ref, d_ref):
    h = jnp.dot(x_ref[...], w_ref[...], preferred_element_type=_f32)
    h_ref[...] = h.astype(_bf16)
    s_ref[...] = jnp.dot(h, es_ref[...], preferred_element_type=_f32)
    d_ref[...] = jnp.dot(h, ed_ref[...], preferred_element_type=_f32)


def _tc2_body(p0_ref, p1_ref, b1_ref, w2_ref, es_ref, ed_ref, r_ref,
              g_ref, s_ref, d_ref):
    acc = p0_ref[...] + p1_ref[...]
    num = acc[:, :HID]
    den = acc[:, HID:HID + HEADS]
    dexp = jnp.dot(den, r_ref[...], preferred_element_type=_f32,
                   precision=lax.Precision.HIGHEST)
    o1 = num / (dexp + 1e-16) + b1_ref[...]
    h2 = jnp.where(o1 > 0.0, o1, jnp.exp(o1) - 1.0)
    g = jnp.dot(h2, w2_ref[...], preferred_element_type=_f32)
    g_ref[...] = g
    s_ref[...] = jnp.dot(g, es_ref[...], preferred_element_type=_f32)
    d_ref[...] = jnp.dot(g, ed_ref[...], preferred_element_type=_f32)


def _tc3_body(q0_ref, q1_ref, b2_ref, o_ref):
    acc = q0_ref[...] + q1_ref[...]
    den = acc[:, GW:GW + 1]
    o = acc / (den + 1e-16) + b2_ref[...]
    col = lax.broadcasted_iota(_i32, (RB, W2ACC), 1)
    valid = col < NCLS
    ov = jnp.where(valid, o, -1e30)
    m = jnp.max(ov, axis=1, keepdims=True)
    ex = jnp.where(valid, jnp.exp(ov - m), 0.0)
    ssum = jnp.sum(ex, axis=1, keepdims=True)
    o_ref[...] = (o - m) - jnp.log(ssum)


def _edge_weight(a):
    a = jnp.where(a >= 0.0, a, 0.2 * a)
    return jnp.exp(a)


def _edge1(tb, db, msgb, b, e):
    sv = plsc.bitcast(tb[b, e, pl.ds(HID, 32)], _f32)
    a = sv + db[b, e, :]
    w = _edge_weight(a)
    pairs = [
        plsc.unpack(plsc.bitcast(tb[b, e, pl.ds(32 * p, 32)], _bf16),
                    format=plsc.PackFormat.INTERLEAVED)
        for p in range(4)
    ]
    msgb[b, e, pl.ds(HID, 16)] = w
    prods = []
    for p in range(4):
        fa, fb = pairs[p]
        prods.append(fa * w[2 * p])
        prods.append(fb * w[2 * p + 1])
    for f in range(8):
        msgb[b, e, pl.ds(f * 16, 16)] = prods[f]


def _edge2(tb, db, msgb, b, e):
    sv = plsc.bitcast(tb[b, e, pl.ds(64, 32)], _f32)
    a = sv + db[b, e, :]
    w = _edge_weight(a)
    fa, fb = plsc.unpack(plsc.bitcast(tb[b, e, pl.ds(0, 32)], _bf16),
                         format=plsc.PackFormat.INTERLEAVED)
    f2 = plsc.bitcast(tb[b, e, pl.ds(32, 32)], _f32)
    msgb[b, e, pl.ds(GW, 16)] = w
    pa = fa * w[0]
    pb = fb * w[0]
    pc = f2 * w[0]
    msgb[b, e, pl.ds(0, 16)] = pa
    msgb[b, e, pl.ds(16, 16)] = pb
    msgb[b, e, pl.ds(32, 16)] = pc


def _make_sc_body(CC, KK, W, edge_fn):
    ZR = ROWS_PT // 8

    def body(sdidx_h, tab, dtab, out,
             idxb, tb, db, msgb, accum, isem, gs0, gs1, ss0, ss1):
        cid = lax.axis_index("c")
        sid = lax.axis_index("s")
        t = cid * NSUB + sid
        gsem = (gs0, gs1)
        ssem = (ss0, ss1)

        def zrow(r, c2):
            for ccol in range(W // 16):
                msgb[0, r, pl.ds(ccol * 16, 16)] = jnp.zeros((16,), _f32)
            return c2

        lax.fori_loop(0, ZR, zrow, 0)
        for i in range(8):
            pltpu.sync_copy(msgb.at[0, pl.ds(0, ZR)],
                            accum.at[pl.ds(sid * ROWS_PT + i * ZR, ZR)])
        pltpu.sync_copy(sdidx_h.at[t, 0], idxb.at[0])
        pltpu.sync_copy(sdidx_h.at[t, 1], idxb.at[1])
        plsc.subcore_barrier()

        def issue_gathers(islot, dslot):
            pltpu.async_copy(tab.at[idxb.at[islot, 0]], tb.at[dslot],
                             gsem[dslot])
            pltpu.async_copy(dtab.at[idxb.at[islot, 1]], db.at[dslot],
                             gsem[dslot])

        issue_gathers(0, 0)

        def outer(qo, carry):
            for q in range(4):
                j = qo * 4 + q
                b = q & 1

                @pl.when((j >= 1) & (j <= KK - 2))
                def _():
                    pltpu.make_async_copy(sdidx_h.at[t, 0],
                                          idxb.at[(q + 1) % 4], isem).wait()

                @pl.when(j <= KK - 2)
                def _():
                    issue_gathers((q + 1) % 4, (q + 1) & 1)

                pltpu.make_async_copy(tab.at[idxb.at[q, 0]], tb.at[b],
                                      gsem[b]).wait()
                pltpu.make_async_copy(dtab.at[idxb.at[q, 1]], db.at[b],
                                      gsem[b]).wait()

                @pl.when(j >= 2)
                def _():
                    pltpu.make_async_copy(
                        msgb.at[b], accum.at[idxb.at[(q + 2) % 4, 1]],
                        ssem[b]).wait()

                @pl.when(j <= KK - 3)
                def _():
                    pltpu.async_copy(sdidx_h.at[t, j + 2],
                                     idxb.at[(q + 2) % 4], isem)

                def edge(e, c2):
                    edge_fn(tb, db, msgb, b, e)
                    return c2

                lax.fori_loop(0, CC, edge, 0, unroll=4)
                pltpu.async_copy(msgb.at[b], accum.at[idxb.at[q, 1]],
                                 ssem[b], add=True)
            return carry

        lax.fori_loop(0, KK // 4, outer, 0)
        pltpu.make_async_copy(msgb.at[0], accum.at[idxb.at[(KK - 2) % 4, 1]],
                              ss0).wait()
        pltpu.make_async_copy(msgb.at[1], accum.at[idxb.at[(KK - 1) % 4, 1]],
                              ss1).wait()
        plsc.subcore_barrier()
        pltpu.sync_copy(accum.at[pl.ds(sid * ROWS_PT, ROWS_PT)],
                        out.at[cid, pl.ds(sid * ROWS_PT, ROWS_PT)])

    return body


_sc1_body = _make_sc_body(C1, K1, W1ACC, _edge1)
_sc2_body = _make_sc_body(C2, K2, W2ACC, _edge2)


def _sc_mesh():
    return plsc.VectorSubcoreMesh(core_axis_name="c", subcore_axis_name="s",
                                  num_cores=2, num_subcores=NSUB)


def _sc_params():
    return pltpu.CompilerParams(needs_layout_passes=False,
                                use_tc_tiling_on_sc=False)


def _sc_scratch(CC, TW, W):
    return [
        pltpu.VMEM((4, 2, CC), _i32),
        pltpu.VMEM((2, CC, TW), _i16),
        pltpu.VMEM((2, CC, 16), _f32),
        pltpu.VMEM((2, CC, W), _f32),
        pltpu.VMEM_SHARED((NA, W), _f32),
        pltpu.SemaphoreType.DMA,
        pltpu.SemaphoreType.DMA,
        pltpu.SemaphoreType.DMA,
        pltpu.SemaphoreType.DMA,
        pltpu.SemaphoreType.DMA,
    ]


def kernel(x, edge_index, W1, a_src1, a_dst1, b1, W2, a_src2, a_dst2, b2):
    loops = jnp.arange(N, dtype=_i32)
    src = jnp.concatenate([edge_index[0].astype(_i32), loops])
    dst = jnp.concatenate([edge_index[1].astype(_i32), loops])
    padlen = ET_PAD - src.shape[0]
    pad = jnp.full((padlen,), N, _i32)
    src_f = jnp.concatenate([src, pad])
    dst_f = jnp.concatenate([dst, pad])
    sd1 = jnp.stack([src_f.reshape(NW, K1, C1),
                     dst_f.reshape(NW, K1, C1)], axis=2)
    sd2 = jnp.stack([src_f.reshape(NW, K2, C2),
                     dst_f.reshape(NW, K2, C2)], axis=2)

    x_pad = jnp.zeros((NA, FIN), _f32).at[:N].set(x)

    hh = jnp.arange(HID, dtype=_i32) // OC
    kk = jnp.arange(16, dtype=_i32) % HEADS
    k8 = jnp.arange(HEADS, dtype=_i32)
    es1 = jnp.where(hh[:, None] == kk[None, :], a_src1.reshape(-1)[:, None], 0.0)
    ed1 = jnp.where(hh[:, None] == kk[None, :], a_dst1.reshape(-1)[:, None], 0.0)
    rmat = (hh[None, :] == k8[:, None]).astype(_f32)
    w2p = jnp.zeros((HID, GW), _f32).at[:, :NCLS].set(W2)
    a2s = jnp.zeros((GW,), _f32).at[:NCLS].set(a_src2[0])
    a2d = jnp.zeros((GW,), _f32).at[:NCLS].set(a_dst2[0])
    es2 = jnp.broadcast_to(a2s[:, None], (GW, 16))
    ed2 = jnp.broadcast_to(a2d[:, None], (GW, 16))
    b1r = b1.reshape(1, HID)
    b2r = jnp.zeros((1, W2ACC), _f32).at[0, :NCLS].set(b2)

    grid = (NA // RB,)
    rep = lambda i: (0, 0)
    row = lambda i: (i, 0)

    hb1, s1, dd1 = pl.pallas_call(
        _tc1_body,
        grid=grid,
        in_specs=[pl.BlockSpec((RB, FIN), row),
                  pl.BlockSpec((FIN, HID), rep),
                  pl.BlockSpec((HID, 16), rep),
                  pl.BlockSpec((HID, 16), rep)],
        out_specs=[pl.BlockSpec((RB, HID), row),
                   pl.BlockSpec((RB, 16), row),
                   pl.BlockSpec((RB, 16), row)],
        out_shape=[jax.ShapeDtypeStruct((NA, HID), _bf16),
                   jax.ShapeDtypeStruct((NA, 16), _f32),
                   jax.ShapeDtypeStruct((NA, 16), _f32)],
    )(x_pad, W1, es1, ed1)

    hi1 = hb1.reshape(NA, 4, 2, 16).transpose(0, 1, 3, 2).reshape(NA, HID)
    tab1 = jnp.concatenate(
        [lax.bitcast_convert_type(hi1, _i16),
         lax.bitcast_convert_type(s1, _i16).reshape(NA, 32)], axis=1)

    parts1 = pl.kernel(
        _sc1_body,
        out_type=jax.ShapeDtypeStruct((2, NA, W1ACC), _f32),
        mesh=_sc_mesh(),
        scratch_types=_sc_scratch(C1, TW1, W1ACC),
        compiler_params=_sc_params(),
    )(sd1, tab1, dd1)

    g2, s2, dd2 = pl.pallas_call(
        _tc2_body,
        grid=grid,
        in_specs=[pl.BlockSpec((RB, W1ACC), row),
                  pl.BlockSpec((RB, W1ACC), row),
                  pl.BlockSpec((1, HID), rep),
                  pl.BlockSpec((FIN, GW), rep),
                  pl.BlockSpec((GW, 16), rep),
                  pl.BlockSpec((GW, 16), rep),
                  pl.BlockSpec((HEADS, HID), rep)],
        out_specs=[pl.BlockSpec((RB, GW), row),
                   pl.BlockSpec((RB, 16), row),
                   pl.BlockSpec((RB, 16), row)],
        out_shape=[jax.ShapeDtypeStruct((NA, GW), _f32),
                   jax.ShapeDtypeStruct((NA, 16), _f32),
                   jax.ShapeDtypeStruct((NA, 16), _f32)],
    )(parts1[0], parts1[1], b1r, w2p, es2, ed2, rmat)

    gb = g2[:, :32].astype(_bf16)
    gi = gb.reshape(NA, 1, 2, 16).transpose(0, 1, 3, 2).reshape(NA, 32)
    tab2 = jnp.concatenate(
        [lax.bitcast_convert_type(gi, _i16),
         lax.bitcast_convert_type(g2[:, 32:GW], _i16).reshape(NA, 32),
         lax.bitcast_convert_type(s2, _i16).reshape(NA, 32)], axis=1)

    parts2 = pl.kernel(
        _sc2_body,
        out_type=jax.ShapeDtypeStruct((2, NA, W2ACC), _f32),
        mesh=_sc_mesh(),
        scratch_types=_sc_scratch(C2, TW2, W2ACC),
        compiler_params=_sc_params(),
    )(sd2, tab2, dd2)

    res = pl.pallas_call(
        _tc3_body,
        grid=grid,
        in_specs=[pl.BlockSpec((RB, W2ACC), row),
                  pl.BlockSpec((RB, W2ACC), row),
                  pl.BlockSpec((1, W2ACC), rep)],
        out_specs=pl.BlockSpec((RB, W2ACC), row),
        out_shape=jax.ShapeDtypeStruct((NA, W2ACC), _f32),
    )(parts2[0], parts2[1], b2r)

    return res[:N, :NCLS]


# speedup vs baseline: 1.1074x; 1.0438x over previous
import jax
import jax.numpy as jnp
from jax import lax
from jax.experimental import pallas as pl
from jax.experimental.pallas import tpu as pltpu
from jax.experimental.pallas import tpu_sc as plsc

N = 10000
FIN = 128
HEADS = 8
OC = 16
HID = HEADS * OC
NCLS = 40

NA = 10112
RB = 128
NW = 32
NSUB = 16
ROWS_PT = NA // NSUB

C1 = 80
K1 = 132
C2 = 120
K2 = 88
T = C1 * K1
ET_PAD = NW * T

W1ACC = HID + 16
TW1 = 160
GW = 48
W2ACC = 64
TW2 = 96

_f32 = jnp.float32
_i32 = jnp.int32
_i16 = jnp.int16
_bf16 = jnp.bfloat16


def _tc1_body(x_ref, w_ref, es_ref, ed_ref, h_ref, s_ref, d_ref):
    h = jnp.dot(x_ref[...], w_ref[...], preferred_element_type=_f32)
    h_ref[...] = h.astype(_bf16)
    s_ref[...] = jnp.dot(h, es_ref[...], preferred_element_type=_f32)
    d_ref[...] = jnp.dot(h, ed_ref[...], preferred_element_type=_f32)


def _tc2_body(p0_ref, p1_ref, b1_ref, w2_ref, es_ref, ed_ref, r_ref,
              g_ref, s_ref, d_ref):
    acc = p0_ref[...] + p1_ref[...]
    num = acc[:, :HID]
    den = acc[:, HID:HID + HEADS]
    dexp = jnp.dot(den, r_ref[...], preferred_element_type=_f32,
                   precision=lax.Precision.HIGHEST)
    o1 = num / (dexp + 1e-16) + b1_ref[...]
    h2 = jnp.where(o1 > 0.0, o1, jnp.exp(o1) - 1.0)
    g = jnp.dot(h2, w2_ref[...], preferred_element_type=_f32)
    g_ref[...] = g
    s_ref[...] = jnp.dot(g, es_ref[...], preferred_element_type=_f32)
    d_ref[...] = jnp.dot(g, ed_ref[...], preferred_element_type=_f32)


def _tc3_body(q0_ref, q1_ref, b2_ref, o_ref):
    acc = q0_ref[...] + q1_ref[...]
    den = acc[:, GW:GW + 1]
    o = acc / (den + 1e-16) + b2_ref[...]
    col = lax.broadcasted_iota(_i32, (RB, W2ACC), 1)
    valid = col < NCLS
    ov = jnp.where(valid, o, -1e30)
    m = jnp.max(ov, axis=1, keepdims=True)
    ex = jnp.where(valid, jnp.exp(ov - m), 0.0)
    ssum = jnp.sum(ex, axis=1, keepdims=True)
    o_ref[...] = (o - m) - jnp.log(ssum)


def _edge_weight(a):
    a = jnp.where(a >= 0.0, a, 0.2 * a)
    return jnp.exp(a)


def _edge1(tb, db, msgb, b, e):
    sv = plsc.bitcast(tb[b, e, pl.ds(HID, 32)], _f32)
    a = sv + db[b, e, :]
    w = _edge_weight(a)
    pairs = [
        plsc.unpack(plsc.bitcast(tb[b, e, pl.ds(32 * p, 32)], _bf16),
                    format=plsc.PackFormat.INTERLEAVED)
        for p in range(4)
    ]
    msgb[b, e, pl.ds(HID, 16)] = w
    prods = []
    for p in range(4):
        fa, fb = pairs[p]
        prods.append(fa * w[2 * p])
        prods.append(fb * w[2 * p + 1])
    for f in range(8):
        msgb[b, e, pl.ds(f * 16, 16)] = prods[f]


def _edge2(tb, db, msgb, b, e):
    sv = plsc.bitcast(tb[b, e, pl.ds(64, 32)], _f32)
    a = sv + db[b, e, :]
    w = _edge_weight(a)
    fa, fb = plsc.unpack(plsc.bitcast(tb[b, e, pl.ds(0, 32)], _bf16),
                         format=plsc.PackFormat.INTERLEAVED)
    f2 = plsc.bitcast(tb[b, e, pl.ds(32, 32)], _f32)
    msgb[b, e, pl.ds(GW, 16)] = w
    pa = fa * w[0]
    pb = fb * w[0]
    pc = f2 * w[0]
    msgb[b, e, pl.ds(0, 16)] = pa
    msgb[b, e, pl.ds(16, 16)] = pb
    msgb[b, e, pl.ds(32, 16)] = pc


def _make_sc_body(CC, KK, W, edge_fn):
    ZR = ROWS_PT // 8

    def body(sdidx_h, tab, dtab, out,
             idxb, tb, db, msgb, accum, isem, gs0, gs1, ss0, ss1):
        cid = lax.axis_index("c")
        sid = lax.axis_index("s")
        t = cid * NSUB + sid
        gsem = (gs0, gs1)
        ssem = (ss0, ss1)

        def zrow(r, c2):
            for ccol in range(W // 16):
                msgb[0, r, pl.ds(ccol * 16, 16)] = jnp.zeros((16,), _f32)
            return c2

        lax.fori_loop(0, ZR, zrow, 0)
        for i in range(8):
            pltpu.sync_copy(msgb.at[0, pl.ds(0, ZR)],
                            accum.at[pl.ds(sid * ROWS_PT + i * ZR, ZR)])
        pltpu.sync_copy(sdidx_h.at[t, 0], idxb.at[0])
        pltpu.sync_copy(sdidx_h.at[t, 1], idxb.at[1])
        plsc.subcore_barrier()

        def issue_gathers(islot, dslot):
            pltpu.async_copy(tab.at[idxb.at[islot, 0]], tb.at[dslot],
                             gsem[dslot])
            pltpu.async_copy(dtab.at[idxb.at[islot, 1]], db.at[dslot],
                             gsem[dslot])

        issue_gathers(0, 0)

        def outer(qo, carry):
            for q in range(4):
                j = qo * 4 + q
                b = q & 1

                @pl.when((j >= 1) & (j <= KK - 2))
                def _():
                    pltpu.make_async_copy(sdidx_h.at[t, 0],
                                          idxb.at[(q + 1) % 4], isem).wait()

                @pl.when(j <= KK - 2)
                def _():
                    issue_gathers((q + 1) % 4, (q + 1) & 1)

                pltpu.make_async_copy(tab.at[idxb.at[q, 0]], tb.at[b],
                                      gsem[b]).wait()
                pltpu.make_async_copy(dtab.at[idxb.at[q, 1]], db.at[b],
                                      gsem[b]).wait()

                @pl.when(j >= 2)
                def _():
                    pltpu.make_async_copy(
                        msgb.at[b], accum.at[idxb.at[(q + 2) % 4, 1]],
                        ssem[b]).wait()

                @pl.when(j <= KK - 3)
                def _():
                    pltpu.async_copy(sdidx_h.at[t, j + 2],
                                     idxb.at[(q + 2) % 4], isem)

                def edge(e, c2):
                    edge_fn(tb, db, msgb, b, e)
                    return c2

                lax.fori_loop(0, CC, edge, 0, unroll=4)
                pltpu.async_copy(msgb.at[b], accum.at[idxb.at[q, 1]],
                                 ssem[b], add=True)
            return carry

        lax.fori_loop(0, KK // 4, outer, 0)
        pltpu.make_async_copy(msgb.at[0], accum.at[idxb.at[(KK - 2) % 4, 1]],
                              ss0).wait()
        pltpu.make_async_copy(msgb.at[1], accum.at[idxb.at[(KK - 1) % 4, 1]],
                              ss1).wait()
        plsc.subcore_barrier()
        pltpu.sync_copy(accum.at[pl.ds(sid * ROWS_PT, ROWS_PT)],
                        out.at[cid, pl.ds(sid * ROWS_PT, ROWS_PT)])

    return body


_sc1_body = _make_sc_body(C1, K1, W1ACC, _edge1)
_sc2_body = _make_sc_body(C2, K2, W2ACC, _edge2)


def _sc_mesh():
    return plsc.VectorSubcoreMesh(core_axis_name="c", subcore_axis_name="s",
                                  num_cores=2, num_subcores=NSUB)


def _sc_params():
    return pltpu.CompilerParams(needs_layout_passes=False,
                                use_tc_tiling_on_sc=False)


def _sc_scratch(CC, TW, W):
    return [
        pltpu.VMEM((4, 2, CC), _i32),
        pltpu.VMEM((2, CC, TW), _i16),
        pltpu.VMEM((2, CC, 16), _f32),
        pltpu.VMEM((2, CC, W), _f32),
        pltpu.VMEM_SHARED((NA, W), _f32),
        pltpu.SemaphoreType.DMA,
        pltpu.SemaphoreType.DMA,
        pltpu.SemaphoreType.DMA,
        pltpu.SemaphoreType.DMA,
        pltpu.SemaphoreType.DMA,
    ]


def kernel(x, edge_index, W1, a_src1, a_dst1, b1, W2, a_src2, a_dst2, b2):
    loops = jnp.arange(N, dtype=_i32)
    src = jnp.concatenate([edge_index[0].astype(_i32), loops])
    dst = jnp.concatenate([edge_index[1].astype(_i32), loops])
    padlen = ET_PAD - src.shape[0]
    pad = jnp.full((padlen,), N, _i32)
    src_f = jnp.concatenate([src, pad])
    dst_f = jnp.concatenate([dst, pad])
    sd1 = jnp.stack([src_f.reshape(NW, K1, C1),
                     dst_f.reshape(NW, K1, C1)], axis=2)
    sd2 = jnp.stack([src_f.reshape(NW, K2, C2),
                     dst_f.reshape(NW, K2, C2)], axis=2)

    x_pad = jnp.zeros((NA, FIN), _f32).at[:N].set(x)

    hh = jnp.arange(HID, dtype=_i32) // OC
    kk = jnp.arange(16, dtype=_i32) % HEADS
    k8 = jnp.arange(HEADS, dtype=_i32)
    es1 = jnp.where(hh[:, None] == kk[None, :], a_src1.reshape(-1)[:, None], 0.0)
    ed1 = jnp.where(hh[:, None] == kk[None, :], a_dst1.reshape(-1)[:, None], 0.0)
    rmat = (hh[None, :] == k8[:, None]).astype(_f32)
    perm1 = [32 * (l // 32) + 16 * (l % 2) + (l % 32) // 2 for l in range(HID)]
    perm2 = [16 * (l % 2) + l // 2 for l in range(32)] + list(range(32, GW))
    W1p = W1[:, jnp.array(perm1)]
    es1 = es1[jnp.array(perm1), :]
    ed1 = ed1[jnp.array(perm1), :]
    w2p = jnp.zeros((HID, GW), _f32).at[:, :NCLS].set(W2)
    w2p = w2p[:, jnp.array(perm2)]
    a2s = jnp.zeros((GW,), _f32).at[:NCLS].set(a_src2[0])[jnp.array(perm2)]
    a2d = jnp.zeros((GW,), _f32).at[:NCLS].set(a_dst2[0])[jnp.array(perm2)]
    es2 = jnp.broadcast_to(a2s[:, None], (GW, 16))
    ed2 = jnp.broadcast_to(a2d[:, None], (GW, 16))
    b1r = b1.reshape(1, HID)
    b2r = jnp.zeros((1, W2ACC), _f32).at[0, :NCLS].set(b2)

    grid = (NA // RB,)
    rep = lambda i: (0, 0)
    row = lambda i: (i, 0)

    hb1, s1, dd1 = pl.pallas_call(
        _tc1_body,
        grid=grid,
        in_specs=[pl.BlockSpec((RB, FIN), row),
                  pl.BlockSpec((FIN, HID), rep),
                  pl.BlockSpec((HID, 16), rep),
                  pl.BlockSpec((HID, 16), rep)],
        out_specs=[pl.BlockSpec((RB, HID), row),
                   pl.BlockSpec((RB, 16), row),
                   pl.BlockSpec((RB, 16), row)],
        out_shape=[jax.ShapeDtypeStruct((NA, HID), _bf16),
                   jax.ShapeDtypeStruct((NA, 16), _f32),
                   jax.ShapeDtypeStruct((NA, 16), _f32)],
    )(x_pad, W1p, es1, ed1)

    tab1 = jnp.concatenate(
        [lax.bitcast_convert_type(hb1, _i16),
         lax.bitcast_convert_type(s1, _i16).reshape(NA, 32)], axis=1)

    parts1 = pl.kernel(
        _sc1_body,
        out_type=jax.ShapeDtypeStruct((2, NA, W1ACC), _f32),
        mesh=_sc_mesh(),
        scratch_types=_sc_scratch(C1, TW1, W1ACC),
        compiler_params=_sc_params(),
    )(sd1, tab1, dd1)

    g2, s2, dd2 = pl.pallas_call(
        _tc2_body,
        grid=grid,
        in_specs=[pl.BlockSpec((RB, W1ACC), row),
                  pl.BlockSpec((RB, W1ACC), row),
                  pl.BlockSpec((1, HID), rep),
                  pl.BlockSpec((FIN, GW), rep),
                  pl.BlockSpec((GW, 16), rep),
                  pl.BlockSpec((GW, 16), rep),
                  pl.BlockSpec((HEADS, HID), rep)],
        out_specs=[pl.BlockSpec((RB, GW), row),
                   pl.BlockSpec((RB, 16), row),
                   pl.BlockSpec((RB, 16), row)],
        out_shape=[jax.ShapeDtypeStruct((NA, GW), _f32),
                   jax.ShapeDtypeStruct((NA, 16), _f32),
                   jax.ShapeDtypeStruct((NA, 16), _f32)],
    )(parts1[0], parts1[1], b1r, w2p, es2, ed2, rmat)

    tab2 = jnp.concatenate(
        [lax.bitcast_convert_type(g2[:, :32].astype(_bf16), _i16),
         lax.bitcast_convert_type(g2[:, 32:GW], _i16).reshape(NA, 32),
         lax.bitcast_convert_type(s2, _i16).reshape(NA, 32)], axis=1)

    parts2 = pl.kernel(
        _sc2_body,
        out_type=jax.ShapeDtypeStruct((2, NA, W2ACC), _f32),
        mesh=_sc_mesh(),
        scratch_types=_sc_scratch(C2, TW2, W2ACC),
        compiler_params=_sc_params(),
    )(sd2, tab2, dd2)

    res = pl.pallas_call(
        _tc3_body,
        grid=grid,
        in_specs=[pl.BlockSpec((RB, W2ACC), row),
                  pl.BlockSpec((RB, W2ACC), row),
                  pl.BlockSpec((1, W2ACC), rep)],
        out_specs=pl.BlockSpec((RB, W2ACC), row),
        out_shape=jax.ShapeDtypeStruct((NA, W2ACC), _f32),
    )(parts2[0], parts2[1], b2r)

    return res[:N, :NCLS]
